# Initial kernel scaffold; baseline (speedup 1.0000x reference)
import jax
import jax.numpy as jnp
from jax.experimental import pallas as pl


def kernel(x, W1, b1, W2, b2, fc1_W, fc1_b, fc2_W, fc2_b):
    raise NotImplementedError("write your pallas kernel here")


# speedup vs baseline: 200.2879x; 200.2879x over previous
import numpy as np
import jax
import jax.numpy as jnp
from jax.experimental import pallas as pl

_B, _S, _F, _C = 8, 32, 3, 64
_H = 256
_NS = 250
_G = _B * _C
_N = _G * _S
_GB = 64
_R = _GB * _S
_CH = 128
_NCH = _R // _CH


def _make_bd():
    dinv = 1.0 / np.sqrt(np.arange(1, _S + 1, dtype=np.float64))
    m = np.tril(np.outer(dinv, dinv))
    bd = np.zeros((_CH, _CH), np.float64)
    for t in range(_CH // _S):
        bd[t * _S:(t + 1) * _S, t * _S:(t + 1) * _S] = m
    return jnp.asarray(bd, jnp.float32)


_BD = _make_bd()


def _gnn_body(x_ref, w1_ref, b1_ref, w2_ref, b2_ref, bd_ref, out_ref):
    bd = bd_ref[...]
    b1 = b1_ref[...]
    b2 = b2_ref[...]
    a = jnp.dot(x_ref[...], w1_ref[...], preferred_element_type=jnp.float32)
    m1 = jnp.concatenate(
        [jnp.dot(bd, a[t * _CH:(t + 1) * _CH, :],
                 preferred_element_type=jnp.float32) for t in range(_NCH)],
        axis=0)
    h1 = jnp.maximum(m1 + b1, 0.0)
    p2 = jnp.dot(h1, w2_ref[...], preferred_element_type=jnp.float32)
    m2 = jnp.concatenate(
        [jnp.dot(bd, p2[t * _CH:(t + 1) * _CH, :],
                 preferred_element_type=jnp.float32) for t in range(_NCH)],
        axis=0)
    h2 = jnp.maximum(m2 + b2, 0.0)
    out_ref[...] = h2.reshape(_GB, _S, _H).sum(axis=1) * (1.0 / _S)


def _head_body(p_ref, fc1w_ref, fc1b_ref, fc2w_ref, fc2b_ref, out_ref):
    p = p_ref[...].reshape(_B, _C, _H).sum(axis=1) * (1.0 / _C)
    h = jnp.maximum(
        jnp.dot(p, fc1w_ref[...], preferred_element_type=jnp.float32)
        + fc1b_ref[...], 0.0)
    out_ref[...] = (jnp.dot(h, fc2w_ref[...], preferred_element_type=jnp.float32)
                    + fc2b_ref[...])


def kernel(x, W1, b1, W2, b2, fc1_W, fc1_b, fc2_W, fc2_b):
    xt = jnp.transpose(x, (0, 3, 1, 2)).reshape(_N, _F)
    pooled = pl.pallas_call(
        _gnn_body,
        grid=(_G // _GB,),
        in_specs=[
            pl.BlockSpec((_R, _F), lambda i: (i, 0)),
            pl.BlockSpec((_F, _H), lambda i: (0, 0)),
            pl.BlockSpec((1, _H), lambda i: (0, 0)),
            pl.BlockSpec((_H, _H), lambda i: (0, 0)),
            pl.BlockSpec((1, _H), lambda i: (0, 0)),
            pl.BlockSpec((_CH, _CH), lambda i: (0, 0)),
        ],
        out_specs=pl.BlockSpec((_GB, _H), lambda i: (i, 0)),
        out_shape=jax.ShapeDtypeStruct((_G, _H), jnp.float32),
    )(xt, W1, b1.reshape(1, _H), W2, b2.reshape(1, _H), _BD)
    return pl.pallas_call(
        _head_body,
        out_shape=jax.ShapeDtypeStruct((_B, _NS), jnp.float32),
    )(pooled, fc1_W, fc1_b.reshape(1, _H), fc2_W, fc2_b.reshape(1, _NS))


# speedup vs baseline: 223.2322x; 1.1146x over previous
import numpy as np
import jax
import jax.numpy as jnp
from jax.experimental import pallas as pl
from jax.experimental.pallas import tpu as pltpu

_B, _S, _F, _C = 8, 32, 3, 64
_H = 256
_NS = 250
_G = _B * _C
_N = _G * _S
_GB = 64
_R = _GB * _S
_CH = 128
_NCH = _R // _CH


def _make_bd():
    dinv = 1.0 / np.sqrt(np.arange(1, _S + 1, dtype=np.float64))
    m = np.tril(np.outer(dinv, dinv))
    bd = np.zeros((_CH, _CH), np.float64)
    for t in range(_CH // _S):
        bd[t * _S:(t + 1) * _S, t * _S:(t + 1) * _S] = m
    return jnp.asarray(bd, jnp.float32)


_BD = _make_bd()


def _body(x_ref, w1_ref, b1_ref, w2_ref, b2_ref,
          fc1w_ref, fc1b_ref, fc2w_ref, fc2b_ref, bd_ref,
          out_ref, acc_ref):
    i = pl.program_id(0)
    bd = bd_ref[...]
    mx = jnp.concatenate(
        [jnp.dot(bd, x_ref[t * _CH:(t + 1) * _CH, :],
                 preferred_element_type=jnp.float32) for t in range(_NCH)],
        axis=0)
    a = jnp.dot(mx, w1_ref[...], preferred_element_type=jnp.float32)
    h1 = jnp.maximum(a + b1_ref[...], 0.0)
    p2 = jnp.dot(h1, w2_ref[...], preferred_element_type=jnp.float32)
    m2 = jnp.concatenate(
        [jnp.dot(bd, p2[t * _CH:(t + 1) * _CH, :],
                 preferred_element_type=jnp.float32) for t in range(_NCH)],
        axis=0)
    h2 = jnp.maximum(m2 + b2_ref[...], 0.0)
    acc_ref[pl.ds(i, 1), :] = h2.sum(axis=0, keepdims=True) * (1.0 / _R)

    @pl.when(i == _B - 1)
    def _head():
        p = acc_ref[...]
        h = jnp.maximum(
            jnp.dot(p, fc1w_ref[...], preferred_element_type=jnp.float32)
            + fc1b_ref[...], 0.0)
        out_ref[...] = (
            jnp.dot(h, fc2w_ref[...], preferred_element_type=jnp.float32)
            + fc2b_ref[...])


def kernel(x, W1, b1, W2, b2, fc1_W, fc1_b, fc2_W, fc2_b):
    xt = jnp.transpose(x, (0, 3, 1, 2)).reshape(_N, _F)
    return pl.pallas_call(
        _body,
        grid=(_B,),
        in_specs=[
            pl.BlockSpec((_R, _F), lambda i: (i, 0)),
            pl.BlockSpec((_F, _H), lambda i: (0, 0)),
            pl.BlockSpec((1, _H), lambda i: (0, 0)),
            pl.BlockSpec((_H, _H), lambda i: (0, 0)),
            pl.BlockSpec((1, _H), lambda i: (0, 0)),
            pl.BlockSpec((_H, _H), lambda i: (0, 0)),
            pl.BlockSpec((1, _H), lambda i: (0, 0)),
            pl.BlockSpec((_H, _NS), lambda i: (0, 0)),
            pl.BlockSpec((1, _NS), lambda i: (0, 0)),
            pl.BlockSpec((_CH, _CH), lambda i: (0, 0)),
        ],
        out_specs=pl.BlockSpec((_B, _NS), lambda i: (0, 0)),
        out_shape=jax.ShapeDtypeStruct((_B, _NS), jnp.float32),
        scratch_shapes=[pltpu.VMEM((_B, _H), jnp.float32)],
    )(xt, W1, b1.reshape(1, _H), W2, b2.reshape(1, _H),
      fc1_W, fc1_b.reshape(1, _H), fc2_W, fc2_b.reshape(1, _NS), _BD)
